# Initial kernel scaffold; baseline (speedup 1.0000x reference)
import jax
import jax.numpy as jnp
from jax.experimental import pallas as pl


def kernel(token_ids, weight):
    raise NotImplementedError("write your pallas kernel here")


# speedup vs baseline: 1.2849x; 1.2849x over previous
import functools

import jax
import jax.numpy as jnp
from jax import lax
from jax.experimental import pallas as pl
from jax.experimental.pallas import tpu as pltpu
from jax.experimental.pallas import tpu_sc as plsc

_B, _S = 16384, 50
_D = 32
_N_IDX = _B * _S
_CHUNK = 128
_N_ROWS = _N_IDX // _CHUNK

_info = plsc.get_sparse_core_info()
_NC, _NS = _info.num_cores, _info.num_subcores
_NW = _NC * _NS

_ROWS_PER_W = _N_ROWS // _NW
_K = 8
_N_STEPS = _ROWS_PER_W // _K


def _emb_body(table, idx, out, idx_v, rows_v, sem):
    wid = lax.axis_index("s") * _NC + lax.axis_index("c")
    base = wid * _ROWS_PER_W

    def step(g, carry):
        row_off = base + g * _K
        pltpu.sync_copy(idx.at[pl.ds(row_off, _K)], idx_v)
        copies = [
            pltpu.async_copy(table.at[idx_v.at[j]], rows_v.at[j], sem)
            for j in range(_K)
        ]
        for c in copies:
            c.wait()
        pltpu.sync_copy(rows_v, out.at[pl.ds(row_off, _K)])
        return carry

    lax.fori_loop(0, _N_STEPS, step, 0)


@functools.partial(
    pl.kernel,
    mesh=plsc.VectorSubcoreMesh(core_axis_name="c", subcore_axis_name="s"),
    out_type=jax.ShapeDtypeStruct((_N_ROWS, _CHUNK, _D), jnp.float32),
    scratch_types=[
        pltpu.VMEM((_K, _CHUNK), jnp.int32),
        pltpu.VMEM((_K, _CHUNK, _D), jnp.float32),
        pltpu.SemaphoreType.DMA,
    ],
    compiler_params=pltpu.CompilerParams(use_tc_tiling_on_sc=False),
)
def _emb_kernel(table, idx, out, idx_v, rows_v, sem):
    _emb_body(table, idx, out, idx_v, rows_v, sem)


def kernel(token_ids, weight):
    idx = token_ids.astype(jnp.int32).reshape(_N_ROWS, _CHUNK)
    out = _emb_kernel(weight, idx)
    return out.reshape(_B, _S, _D)
